# baseline (speedup 1.0000x reference)
import functools

import jax
import jax.numpy as jnp
from jax import lax
from jax.experimental import pallas as pl
from jax.experimental.pallas import tpu as pltpu
from jax.experimental.pallas import tpu_sc as plsc

EMBED = 64
NUM_CORES = 2
NUM_SUBCORES = 16
NUM_WORKERS = NUM_CORES * NUM_SUBCORES
CHUNK = 1
NBUF = 8


@functools.lru_cache(maxsize=None)
def _build(batch, seqlen):
    rows_per_w = batch // NUM_WORKERS
    n_chunks = rows_per_w // CHUNK
    n_groups = n_chunks // NBUF
    mesh = plsc.VectorSubcoreMesh(core_axis_name="c", subcore_axis_name="s")

    scratch = (
        [pltpu.VMEM((rows_per_w, seqlen), jnp.int32)]
        + [pltpu.VMEM((seqlen, EMBED), jnp.float32) for _ in range(NBUF)]
        + [pltpu.SemaphoreType.DMA for _ in range(2 * NBUF)]
    )

    @functools.partial(
        pl.kernel,
        out_type=jax.ShapeDtypeStruct((batch, seqlen, EMBED), jnp.float32),
        mesh=mesh,
        scratch_types=scratch,
        compiler_params=pltpu.CompilerParams(use_tc_tiling_on_sc=False),
    )
    def gather_kernel(idx_hbm, table_hbm, out_hbm, idx_v, *bufs_and_sems):
        rows = bufs_and_sems[:NBUF]
        gsem = bufs_and_sems[NBUF : 2 * NBUF]
        ssem = bufs_and_sems[2 * NBUF :]
        wid = lax.axis_index("s") * NUM_CORES + lax.axis_index("c")
        base = wid * rows_per_w

        pltpu.sync_copy(idx_hbm.at[pl.ds(base, rows_per_w)], idx_v)

        def start_gather(c, b):
            return pltpu.async_copy(
                table_hbm.at[idx_v.at[c]], rows[b], gsem[b]
            )

        def wait_gather(c, b):
            pltpu.make_async_copy(
                table_hbm.at[idx_v.at[c]], rows[b], gsem[b]
            ).wait()

        def start_scatter(c, b):
            return pltpu.async_copy(
                rows[b], out_hbm.at[base + c], ssem[b]
            )

        def wait_scatter(b):
            pltpu.make_async_copy(
                rows[b], out_hbm.at[base], ssem[b]
            ).wait()

        def group(g, carry):
            gds = []
            for j in range(NBUF):
                @pl.when(g >= 1)
                def _():
                    wait_scatter(j)

                gds.append(start_gather(g * NBUF + j, j))
            for j in range(NBUF):
                gds[j].wait()
                start_scatter(g * NBUF + j, j)
            return carry

        lax.fori_loop(0, n_groups, group, 0)
        for j in range(NBUF):
            wait_scatter(j)

    return gather_kernel


def kernel(sentences, table):
    b, l = sentences.shape
    return _build(b, l)(sentences, table)


# speedup vs baseline: 1.2234x; 1.2234x over previous
import functools

import jax
import jax.numpy as jnp
from jax import lax
from jax.experimental import pallas as pl
from jax.experimental.pallas import tpu as pltpu
from jax.experimental.pallas import tpu_sc as plsc

EMBED = 64
ROW = 128
NUM_CORES = 2
NUM_SUBCORES = 16
NUM_WORKERS = NUM_CORES * NUM_SUBCORES
CHUNK = 200
NBUF = 4


@functools.lru_cache(maxsize=None)
def _build(n_rows):
    per_w = n_rows // NUM_WORKERS
    n_chunks = per_w // CHUNK
    n_groups = n_chunks // NBUF
    mesh = plsc.VectorSubcoreMesh(core_axis_name="c", subcore_axis_name="s")

    scratch = (
        [pltpu.VMEM((CHUNK,), jnp.int32) for _ in range(NBUF)]
        + [pltpu.VMEM((CHUNK, ROW), jnp.float32) for _ in range(NBUF)]
        + [pltpu.SemaphoreType.DMA for _ in range(3 * NBUF)]
    )

    @functools.partial(
        pl.kernel,
        out_type=jax.ShapeDtypeStruct((n_rows, ROW), jnp.float32),
        mesh=mesh,
        scratch_types=scratch,
        compiler_params=pltpu.CompilerParams(use_tc_tiling_on_sc=True),
    )
    def gather_kernel(idx_hbm, table_hbm, out_hbm, *refs):
        idxb = refs[:NBUF]
        rows = refs[NBUF : 2 * NBUF]
        isem = refs[2 * NBUF : 3 * NBUF]
        gsem = refs[3 * NBUF : 4 * NBUF]
        ssem = refs[4 * NBUF :]
        wid = lax.axis_index("s") * NUM_CORES + lax.axis_index("c")
        base = wid * per_w

        def start_idx(c, b):
            return pltpu.async_copy(
                idx_hbm.at[pl.ds(base + c * CHUNK, CHUNK)], idxb[b], isem[b]
            )

        def wait_idx(b):
            pltpu.make_async_copy(
                idx_hbm.at[pl.ds(base, CHUNK)], idxb[b], isem[b]
            ).wait()

        def start_gather(b):
            return pltpu.async_copy(table_hbm.at[idxb[b]], rows[b], gsem[b])

        def start_scatter(c, b):
            return pltpu.async_copy(
                rows[b], out_hbm.at[pl.ds(base + c * CHUNK, CHUNK)], ssem[b]
            )

        def wait_scatter(b):
            pltpu.make_async_copy(
                rows[b], out_hbm.at[pl.ds(base, CHUNK)], ssem[b]
            ).wait()

        for j in range(NBUF):
            start_idx(j, j)

        def group(g, carry):
            gds = []
            for j in range(NBUF):
                @pl.when(g >= 1)
                def _():
                    wait_scatter(j)

                wait_idx(j)
                gds.append(start_gather(j))
            for j in range(NBUF):
                c = g * NBUF + j
                gds[j].wait()
                start_scatter(c, j)

                @pl.when(g <= n_groups - 2)
                def _():
                    start_idx(c + NBUF, j)

            return carry

        lax.fori_loop(0, n_groups, group, 0)
        for j in range(NBUF):
            wait_scatter(j)

    return gather_kernel


def kernel(sentences, table):
    b, l = sentences.shape
    idx = sentences.reshape(b * l)
    table128 = jnp.pad(table, ((0, 0), (0, ROW - EMBED)))
    out = _build(b * l)(idx, table128)
    return out[:, :EMBED].reshape(b, l, EMBED)
